# baseline (speedup 1.0000x reference)
import jax
import jax.numpy as jnp
from jax.experimental import pallas as pl
from jax.experimental.pallas import tpu as pltpu

_B, _K, _D = 64, 1, 1024
_HS, _HR, _E = 4096, 1024, 8
_G = 18
_DC = _D // 8
_SC = _HS // 8
_RH = _HR // 2


def _step(xc_ref, xh_ref, rw_ref, wg_s_ref, bg_s_ref, wu_s_ref, bu_s_ref,
          wd_s_ref, bd_s_ref, wg_r_ref, bg_r_ref, wu_r_ref, bu_r_ref,
          wd_r_ref, bd_r_ref, out_ref,
          g_s, u_s, h_s, g_r, u_r, h_r):
    i = pl.program_id(0)
    d = i % 2

    @pl.when(i == 0)
    def _init_out():
        out_ref[...] = jnp.broadcast_to(bd_s_ref[...][None, :], (_B, _D))

    @pl.when(i >= 2)
    def _routed_down():
        out_ref[...] += jnp.dot(h_r[d], wd_r_ref[0],
                                preferred_element_type=jnp.float32)

    @pl.when((i >= 8) & (i < 16))
    def _shared_down():
        out_ref[...] += jnp.dot(h_s[i - 8], wd_s_ref[...],
                                preferred_element_type=jnp.float32)

    @pl.when(i < 8)
    def _shared_up():
        xk = xc_ref[i]
        pg = jnp.dot(xk, wg_s_ref[...], preferred_element_type=jnp.float32)
        pu = jnp.dot(xk, wu_s_ref[...], preferred_element_type=jnp.float32)

        @pl.when(i == 0)
        def _():
            g_s[...] = pg
            u_s[...] = pu

        @pl.when(i != 0)
        def _():
            g_s[...] += pg
            u_s[...] += pu

    @pl.when(i < 16)
    def _routed_up():
        xd = xh_ref[d]
        pg = jnp.dot(xd, wg_r_ref[0], preferred_element_type=jnp.float32)
        pu = jnp.dot(xd, wu_r_ref[0], preferred_element_type=jnp.float32)

        @pl.when(d == 0)
        def _():
            g_r[...] = pg
            u_r[...] = pu

        @pl.when(d == 1)
        def _():
            gr = g_r[...] + pg + bg_r_ref[0]
            ur = u_r[...] + pu + bu_r_ref[0]
            w = rw_ref[0]
            hr = jax.nn.silu(gr) * ur * w
            h_r[0] = hr[:, :_RH]
            h_r[1] = hr[:, _RH:]
            out_ref[...] += w * bd_r_ref[0]

    @pl.when(i == 7)
    def _shared_act():
        for c in range(8):
            sl = slice(c * _SC, (c + 1) * _SC)
            h_s[c] = (jax.nn.silu(g_s[:, sl] + bg_s_ref[sl])
                      * (u_s[:, sl] + bu_s_ref[sl]))


def kernel(x, routing_weights, Wg_s, bg_s, Wu_s, bu_s, Wd_s, bd_s,
           Wg_r, bg_r, Wu_r, bu_r, Wd_r, bd_r):
    x2 = x.reshape(_B, _D)
    xc = x2.reshape(_B, 8, _DC).transpose(1, 0, 2)
    xh = x2.reshape(_B, 2, _RH).transpose(1, 0, 2)
    rw = routing_weights.T.reshape(_E, _B, 1)
    bg_r3 = bg_r.reshape(_E, 1, _HR)
    bu_r3 = bu_r.reshape(_E, 1, _HR)
    bd_r3 = bd_r.reshape(_E, 1, _D)

    def _ex(i):
        return jnp.minimum(i // 2, _E - 1)

    def _exh(i):
        return jnp.minimum(i // 2, _E - 1), jnp.where(i >= 15, 1, i % 2)

    def _dn(i):
        ii = jnp.clip(i - 2, 0, 15)
        return ii // 2, ii % 2

    out = pl.pallas_call(
        _step,
        grid=(_G,),
        in_specs=[
            pl.BlockSpec((8, _B, _DC), lambda i: (0, 0, 0)),
            pl.BlockSpec((2, _B, _RH), lambda i: (0, 0, 0)),
            pl.BlockSpec((1, _B, 1), lambda i: (_ex(i), 0, 0)),
            pl.BlockSpec((_DC, _HS), lambda i: (jnp.minimum(i, 7), 0)),
            pl.BlockSpec((_HS,), lambda i: (0,)),
            pl.BlockSpec((_DC, _HS), lambda i: (jnp.minimum(i, 7), 0)),
            pl.BlockSpec((_HS,), lambda i: (0,)),
            pl.BlockSpec((_SC, _D),
                         lambda i: (jnp.clip(i - 8, 0, 7), 0)),
            pl.BlockSpec((_D,), lambda i: (0,)),
            pl.BlockSpec((1, _RH, _HR),
                         lambda i: _exh(i) + (0,)),
            pl.BlockSpec((1, 1, _HR), lambda i: (_ex(i), 0, 0)),
            pl.BlockSpec((1, _RH, _HR),
                         lambda i: _exh(i) + (0,)),
            pl.BlockSpec((1, 1, _HR), lambda i: (_ex(i), 0, 0)),
            pl.BlockSpec((1, _RH, _D),
                         lambda i: _dn(i) + (0,)),
            pl.BlockSpec((1, 1, _D), lambda i: (_ex(i), 0, 0)),
        ],
        out_specs=pl.BlockSpec((_B, _D), lambda i: (0, 0)),
        out_shape=jax.ShapeDtypeStruct((_B, _D), jnp.float32),
        scratch_shapes=[
            pltpu.VMEM((_B, _HS), jnp.float32),
            pltpu.VMEM((_B, _HS), jnp.float32),
            pltpu.VMEM((8, _B, _SC), jnp.float32),
            pltpu.VMEM((_B, _HR), jnp.float32),
            pltpu.VMEM((_B, _HR), jnp.float32),
            pltpu.VMEM((2, _B, _RH), jnp.float32),
        ],
        compiler_params=pltpu.CompilerParams(
            dimension_semantics=("arbitrary",),
        ),
    )(xc, xh, rw, Wg_s, bg_s, Wu_s, bu_s, Wd_s, bd_s,
      Wg_r, bg_r3, Wu_r, bu_r3, Wd_r, bd_r3)

    return out.reshape(_B, _K, _D)


# speedup vs baseline: 1.0180x; 1.0180x over previous
import jax
import jax.numpy as jnp
from jax.experimental import pallas as pl
from jax.experimental.pallas import tpu as pltpu

_B, _K, _D = 64, 1, 1024
_HS, _HR, _E = 4096, 1024, 8
_G = 18
_CS = _HS // 16
_RH = _HR // 2


def _step(x_ref, xh_ref, rw_ref, wg_s_ref, bg_s_ref, wu_s_ref, bu_s_ref,
          wd_s_ref, bd_s_ref, wg_r_ref, bg_r_ref, wu_r_ref, bu_r_ref,
          wd_r_ref, bd_r_ref, out_ref, g_r, u_r, h_r):
    i = pl.program_id(0)
    d = i % 2

    @pl.when(i >= 2)
    def _routed_down():
        out_ref[...] += jnp.dot(h_r[d], wd_r_ref[0],
                                preferred_element_type=jnp.float32)

    @pl.when(i < 16)
    def _active():
        xv = x_ref[...]
        g = jnp.dot(xv, wg_s_ref[...], preferred_element_type=jnp.float32)
        u = jnp.dot(xv, wu_s_ref[...], preferred_element_type=jnp.float32)
        h = jax.nn.silu(g + bg_s_ref[...]) * (u + bu_s_ref[...])
        acc = jnp.dot(h, wd_s_ref[...], preferred_element_type=jnp.float32)

        @pl.when(i == 0)
        def _init():
            out_ref[...] = acc + bd_s_ref[...]

        @pl.when(i != 0)
        def _accum():
            out_ref[...] += acc

        xd = xh_ref[d]
        pg = jnp.dot(xd, wg_r_ref[0], preferred_element_type=jnp.float32)
        pu = jnp.dot(xd, wu_r_ref[0], preferred_element_type=jnp.float32)

        @pl.when(d == 0)
        def _first_half():
            g_r[...] = pg
            u_r[...] = pu

        @pl.when(d == 1)
        def _second_half():
            w = rw_ref[0]
            gr = g_r[...] + pg + bg_r_ref[0]
            ur = u_r[...] + pu + bu_r_ref[0]
            hr = jax.nn.silu(gr) * ur * w
            h_r[0] = hr[:, :_RH]
            h_r[1] = hr[:, _RH:]
            out_ref[...] += w * bd_r_ref[0]


def kernel(x, routing_weights, Wg_s, bg_s, Wu_s, bu_s, Wd_s, bd_s,
           Wg_r, bg_r, Wu_r, bu_r, Wd_r, bd_r):
    x2 = x.reshape(_B, _D)
    xh = x2.reshape(_B, 2, _RH).transpose(1, 0, 2)
    rw = routing_weights.T.reshape(_E, _B, 1)
    bg_r3 = bg_r.reshape(_E, 1, _HR)
    bu_r3 = bu_r.reshape(_E, 1, _HR)
    bd_r3 = bd_r.reshape(_E, 1, _D)

    def _sh(i):
        return jnp.minimum(i, 15)

    def _ex(i):
        return jnp.minimum(i // 2, _E - 1)

    def _exh(i):
        return jnp.minimum(i // 2, _E - 1), jnp.where(i >= 15, 1, i % 2)

    def _dn(i):
        ii = jnp.clip(i - 2, 0, 15)
        return ii // 2, ii % 2

    out = pl.pallas_call(
        _step,
        grid=(_G,),
        in_specs=[
            pl.BlockSpec((_B, _D), lambda i: (0, 0)),
            pl.BlockSpec((2, _B, _RH), lambda i: (0, 0, 0)),
            pl.BlockSpec((1, _B, 1), lambda i: (_ex(i), 0, 0)),
            pl.BlockSpec((_D, _CS), lambda i: (0, _sh(i))),
            pl.BlockSpec((_CS,), lambda i: (_sh(i),)),
            pl.BlockSpec((_D, _CS), lambda i: (0, _sh(i))),
            pl.BlockSpec((_CS,), lambda i: (_sh(i),)),
            pl.BlockSpec((_CS, _D), lambda i: (_sh(i), 0)),
            pl.BlockSpec((_D,), lambda i: (0,)),
            pl.BlockSpec((1, _RH, _HR),
                         lambda i: _exh(i) + (0,)),
            pl.BlockSpec((1, 1, _HR), lambda i: (_ex(i), 0, 0)),
            pl.BlockSpec((1, _RH, _HR),
                         lambda i: _exh(i) + (0,)),
            pl.BlockSpec((1, 1, _HR), lambda i: (_ex(i), 0, 0)),
            pl.BlockSpec((1, _RH, _D),
                         lambda i: _dn(i) + (0,)),
            pl.BlockSpec((1, 1, _D), lambda i: (_ex(i), 0, 0)),
        ],
        out_specs=pl.BlockSpec((_B, _D), lambda i: (0, 0)),
        out_shape=jax.ShapeDtypeStruct((_B, _D), jnp.float32),
        scratch_shapes=[
            pltpu.VMEM((_B, _HR), jnp.float32),
            pltpu.VMEM((_B, _HR), jnp.float32),
            pltpu.VMEM((2, _B, _RH), jnp.float32),
        ],
        compiler_params=pltpu.CompilerParams(
            dimension_semantics=("arbitrary",),
        ),
    )(x2, xh, rw, Wg_s, bg_s, Wu_s, bu_s, Wd_s, bd_s,
      Wg_r, bg_r3, Wu_r, bu_r3, Wd_r, bd_r3)

    return out.reshape(_B, _K, _D)


# speedup vs baseline: 1.0537x; 1.0350x over previous
import jax
import jax.numpy as jnp
from jax.experimental import pallas as pl
from jax.experimental.pallas import tpu as pltpu

_B, _K, _D = 64, 1, 1024
_HS, _HR, _E = 4096, 1024, 8
_G = 17
_CS = _HS // 16
_RH = _HR // 2


def _step(x_ref, xh_ref, rw_ref, wg_s_ref, bg_s_ref, wu_s_ref, bu_s_ref,
          wd_s_ref, bd_s_ref, wg_r_ref, bg_r_ref, wu_r_ref, bu_r_ref,
          wd_r_ref, bd_r_ref, out_ref, g_r, u_r, h_r):
    i = pl.program_id(0)
    d = i % 2

    @pl.when(i < 16)
    def _active():
        xv = x_ref[...]
        g = jnp.dot(xv, wg_s_ref[...], preferred_element_type=jnp.float32)
        u = jnp.dot(xv, wu_s_ref[...], preferred_element_type=jnp.float32)
        h = jax.nn.silu(g + bg_s_ref[...]) * (u + bu_s_ref[...])
        acc = jnp.dot(h, wd_s_ref[...], preferred_element_type=jnp.float32)

        @pl.when(i == 0)
        def _init():
            out_ref[...] = acc + bd_s_ref[...]

        @pl.when(i != 0)
        def _accum():
            out_ref[...] += acc

        @pl.when(d == 0)
        def _first_half():
            xd = xh_ref[0]
            g_r[...] = jnp.dot(xd, wg_r_ref[0],
                               preferred_element_type=jnp.float32)
            u_r[...] = jnp.dot(xd, wu_r_ref[0],
                               preferred_element_type=jnp.float32)

        @pl.when(d == 1)
        def _second_half():
            xd = xh_ref[1]
            pg = jnp.dot(xd, wg_r_ref[0], preferred_element_type=jnp.float32)
            pu = jnp.dot(xd, wu_r_ref[0], preferred_element_type=jnp.float32)
            w = rw_ref[0]
            gr = g_r[...] + pg + bg_r_ref[0]
            ur = u_r[...] + pu + bu_r_ref[0]
            hr = jax.nn.silu(gr) * ur * w
            h_r[...] = hr[:, _RH:]
            out_ref[...] += (jnp.dot(hr[:, :_RH], wd_r_ref[0],
                                     preferred_element_type=jnp.float32)
                             + w * bd_r_ref[0])

    @pl.when((i >= 2) & (d == 0))
    def _routed_down_tail():
        out_ref[...] += jnp.dot(h_r[...], wd_r_ref[0],
                                preferred_element_type=jnp.float32)


def kernel(x, routing_weights, Wg_s, bg_s, Wu_s, bu_s, Wd_s, bd_s,
           Wg_r, bg_r, Wu_r, bu_r, Wd_r, bd_r):
    x2 = x.reshape(_B, _D)
    xh = x2.reshape(_B, 2, _RH).transpose(1, 0, 2)
    rw = routing_weights.T.reshape(_E, _B, 1)
    bg_r3 = bg_r.reshape(_E, 1, _HR)
    bu_r3 = bu_r.reshape(_E, 1, _HR)
    bd_r3 = bd_r.reshape(_E, 1, _D)

    def _sh(i):
        return jnp.minimum(i, 15)

    def _ex(i):
        return jnp.minimum(i // 2, _E - 1)

    def _exh(i):
        return jnp.minimum(i // 2, _E - 1), jnp.where(i >= 15, 1, i % 2)

    def _dn(i):
        ii = jnp.clip(i - 1, 0, 15)
        return ii // 2, ii % 2

    out = pl.pallas_call(
        _step,
        grid=(_G,),
        in_specs=[
            pl.BlockSpec((_B, _D), lambda i: (0, 0)),
            pl.BlockSpec((2, _B, _RH), lambda i: (0, 0, 0)),
            pl.BlockSpec((1, _B, 1), lambda i: (_ex(i), 0, 0)),
            pl.BlockSpec((_D, _CS), lambda i: (0, _sh(i))),
            pl.BlockSpec((_CS,), lambda i: (_sh(i),)),
            pl.BlockSpec((_D, _CS), lambda i: (0, _sh(i))),
            pl.BlockSpec((_CS,), lambda i: (_sh(i),)),
            pl.BlockSpec((_CS, _D), lambda i: (_sh(i), 0)),
            pl.BlockSpec((_D,), lambda i: (0,)),
            pl.BlockSpec((1, _RH, _HR),
                         lambda i: _exh(i) + (0,)),
            pl.BlockSpec((1, 1, _HR), lambda i: (_ex(i), 0, 0)),
            pl.BlockSpec((1, _RH, _HR),
                         lambda i: _exh(i) + (0,)),
            pl.BlockSpec((1, 1, _HR), lambda i: (_ex(i), 0, 0)),
            pl.BlockSpec((1, _RH, _D),
                         lambda i: _dn(i) + (0,)),
            pl.BlockSpec((1, 1, _D), lambda i: (_ex(i), 0, 0)),
        ],
        out_specs=pl.BlockSpec((_B, _D), lambda i: (0, 0)),
        out_shape=jax.ShapeDtypeStruct((_B, _D), jnp.float32),
        scratch_shapes=[
            pltpu.VMEM((_B, _HR), jnp.float32),
            pltpu.VMEM((_B, _HR), jnp.float32),
            pltpu.VMEM((_B, _RH), jnp.float32),
        ],
        compiler_params=pltpu.CompilerParams(
            dimension_semantics=("arbitrary",),
        ),
    )(x2, xh, rw, Wg_s, bg_s, Wu_s, bu_s, Wd_s, bd_s,
      Wg_r, bg_r3, Wu_r, bu_r3, Wd_r, bd_r3)

    return out.reshape(_B, _K, _D)


# speedup vs baseline: 1.1011x; 1.0450x over previous
import jax
import jax.numpy as jnp
from jax.experimental import pallas as pl
from jax.experimental.pallas import tpu as pltpu

_B, _K, _D = 64, 1, 1024
_HS, _HR, _E = 4096, 1024, 8
_S = 2
_G = _E * _S
_CS = _HS // _G
_CR = _HR // _S
_DH = _D // 2


def _step(x_ref, rw_ref,
          wg_s_lo, wg_s_hi, bg_s_ref, wu_s_lo, wu_s_hi, bu_s_ref,
          wd_s_lo, wd_s_hi, bd_s_ref,
          wg_r_lo, wg_r_hi, bg_r_ref, wu_r_lo, wu_r_hi, bu_r_ref,
          wd_r_lo, wd_r_hi, bd_r_ref, out_ref):
    i = pl.program_id(0)
    j = i % _S
    xlo = x_ref[:, :_DH]
    xhi = x_ref[:, _DH:]

    g = (jnp.dot(xlo, wg_s_lo[0], preferred_element_type=jnp.float32)
         + jnp.dot(xhi, wg_s_hi[0], preferred_element_type=jnp.float32))
    u = (jnp.dot(xlo, wu_s_lo[0], preferred_element_type=jnp.float32)
         + jnp.dot(xhi, wu_s_hi[0], preferred_element_type=jnp.float32))
    h = jax.nn.silu(g + bg_s_ref[...]) * (u + bu_s_ref[...])
    acc = (jnp.dot(h[:, :_CS // 2], wd_s_lo[...],
                   preferred_element_type=jnp.float32)
           + jnp.dot(h[:, _CS // 2:], wd_s_hi[...],
                     preferred_element_type=jnp.float32))

    w = rw_ref[0]
    gr = (jnp.dot(xlo, wg_r_lo[0, 0], preferred_element_type=jnp.float32)
          + jnp.dot(xhi, wg_r_hi[0, 0], preferred_element_type=jnp.float32))
    ur = (jnp.dot(xlo, wu_r_lo[0, 0], preferred_element_type=jnp.float32)
          + jnp.dot(xhi, wu_r_hi[0, 0], preferred_element_type=jnp.float32))
    hr = jax.nn.silu(gr + bg_r_ref[0]) * (ur + bu_r_ref[0]) * w
    acc = acc + (jnp.dot(hr[:, :_CR // 2], wd_r_lo[0, 0],
                         preferred_element_type=jnp.float32)
                 + jnp.dot(hr[:, _CR // 2:], wd_r_hi[0, 0],
                           preferred_element_type=jnp.float32))
    acc = acc + jnp.where(j == 0, 1.0, 0.0) * (w * bd_r_ref[0])

    @pl.when(i == 0)
    def _init():
        out_ref[...] = acc + bd_s_ref[...]

    @pl.when(i != 0)
    def _accum():
        out_ref[...] += acc


def kernel(x, routing_weights, Wg_s, bg_s, Wu_s, bu_s, Wd_s, bd_s,
           Wg_r, bg_r, Wu_r, bu_r, Wd_r, bd_r):
    x2 = x.reshape(_B, _D)
    rw = routing_weights.T.reshape(_E, _B, 1)
    bg_r3 = bg_r.reshape(_E, 1, _HR)
    bu_r3 = bu_r.reshape(_E, 1, _HR)
    bd_r3 = bd_r.reshape(_E, 1, _D)
    wg_s2 = Wg_s.reshape(2, _DH, _HS)
    wu_s2 = Wu_s.reshape(2, _DH, _HS)
    wg_r2 = Wg_r.reshape(_E, 2, _DH, _HR)
    wu_r2 = Wu_r.reshape(_E, 2, _DH, _HR)
    wd_r2 = Wd_r.reshape(_E, 2 * _S, _CR // 2, _D)

    out = pl.pallas_call(
        _step,
        grid=(_G,),
        in_specs=[
            pl.BlockSpec((_B, _D), lambda i: (0, 0)),
            pl.BlockSpec((1, _B, 1), lambda i: (i // _S, 0, 0)),
            pl.BlockSpec((1, _DH, _CS), lambda i: (0, 0, i)),
            pl.BlockSpec((1, _DH, _CS), lambda i: (1, 0, i)),
            pl.BlockSpec((_CS,), lambda i: (i,)),
            pl.BlockSpec((1, _DH, _CS), lambda i: (0, 0, i)),
            pl.BlockSpec((1, _DH, _CS), lambda i: (1, 0, i)),
            pl.BlockSpec((_CS,), lambda i: (i,)),
            pl.BlockSpec((_CS // 2, _D), lambda i: (2 * i, 0)),
            pl.BlockSpec((_CS // 2, _D), lambda i: (2 * i + 1, 0)),
            pl.BlockSpec((_D,), lambda i: (0,)),
            pl.BlockSpec((1, 1, _DH, _CR),
                         lambda i: (i // _S, 0, 0, i % _S)),
            pl.BlockSpec((1, 1, _DH, _CR),
                         lambda i: (i // _S, 1, 0, i % _S)),
            pl.BlockSpec((1, 1, _CR), lambda i: (i // _S, 0, i % _S)),
            pl.BlockSpec((1, 1, _DH, _CR),
                         lambda i: (i // _S, 0, 0, i % _S)),
            pl.BlockSpec((1, 1, _DH, _CR),
                         lambda i: (i // _S, 1, 0, i % _S)),
            pl.BlockSpec((1, 1, _CR), lambda i: (i // _S, 0, i % _S)),
            pl.BlockSpec((1, 1, _CR // 2, _D),
                         lambda i: (i // _S, 2 * (i % _S), 0, 0)),
            pl.BlockSpec((1, 1, _CR // 2, _D),
                         lambda i: (i // _S, 2 * (i % _S) + 1, 0, 0)),
            pl.BlockSpec((1, 1, _D), lambda i: (i // _S, 0, 0)),
        ],
        out_specs=pl.BlockSpec((_B, _D), lambda i: (0, 0)),
        out_shape=jax.ShapeDtypeStruct((_B, _D), jnp.float32),
        compiler_params=pltpu.CompilerParams(
            dimension_semantics=("arbitrary",),
        ),
    )(x2, rw,
      wg_s2, wg_s2, bg_s, wu_s2, wu_s2, bu_s, Wd_s, Wd_s, bd_s,
      wg_r2, wg_r2, bg_r3, wu_r2, wu_r2, bu_r3, wd_r2, wd_r2, bd_r3)

    return out.reshape(_B, _K, _D)
